# baseline (speedup 1.0000x reference)
import functools
import math

import jax
import jax.numpy as jnp
from jax import lax
from jax.experimental import pallas as pl
from jax.experimental.pallas import tpu as pltpu
from jax.experimental.pallas import tpu_sc as plsc

_SMOOTHING = 0.1
_PAD = 1

_NC = 2
_NS = 16
_NW = _NC * _NS

_RT = 512
_CH = 6400


def _sc_body(lp_hbm, s1_hbm, s0_hbm, buf, obuf1, obuf0, *, n, v):
    wid = lax.axis_index("s") * _NC + lax.axis_index("c")
    rpw = (n - _RT) // _NW
    rbase = _RT + wid * rpw
    va = (v // 128) * 128
    nfull, tail = divmod(va, _CH)

    zero = jnp.zeros((16,), jnp.float32)
    accs = tuple([zero] * rpw)
    first = [zero] * rpw
    for ci in range(nfull + (1 if tail else 0)):
        c0 = ci * _CH
        w = _CH if ci < nfull else tail
        pltpu.sync_copy(
            lp_hbm.at[pl.ds(rbase, rpw), pl.ds(c0, w)],
            buf.at[:, pl.ds(0, w)],
        )

        def body(j, a):
            return tuple(
                a[r] + buf[r, pl.ds(j * 16, 16)] for r in range(rpw)
            )

        accs = lax.fori_loop(0, w // 16, body, accs)
        if ci == 0:
            first = [buf[r, pl.ds(0, 16)] for r in range(rpw)]

    for r in range(rpw):
        obuf1[r, :] = accs[r]
        obuf0[r, :] = first[r]
    row0 = wid * rpw
    pltpu.sync_copy(obuf1, s1_hbm.at[pl.ds(row0, rpw), :])
    pltpu.sync_copy(obuf0, s0_hbm.at[pl.ds(row0, rpw), :])


def _tc_a_body(
    tgt_ref, tsc_s_ref, tsc_ref, lp_ref, strip_ref, lp_any, out_ref,
    gbuf, sem, *, eps, conf, c, v, rb
):
    s = pl.program_id(0)
    blk = lp_ref[...]
    t = tgt_ref[...]
    rowsum = jnp.sum(blk, axis=1, keepdims=True)
    vb = blk[:, _PAD:_PAD + 1]
    cols = lax.broadcasted_iota(jnp.int32, blk.shape, 1)
    vt = jnp.sum(jnp.where(cols == t, blk, 0.0), axis=1, keepdims=True)
    p = jnp.sum(
        jnp.where(
            t != _PAD, c - eps * rowsum + eps * vb + (eps - conf) * vt, 0.0
        )
    )

    va = (v // 128) * 128
    descs = []
    for j in range(rb):
        tj = tsc_s_ref[j, 0]
        start = jnp.minimum((tj >> 7) << 7, va - 128)
        start = pl.multiple_of(start, 128)
        row0 = pl.multiple_of(_RT + s * rb + (j // 8) * 8, 8)
        d = pltpu.make_async_copy(
            lp_any.at[pl.ds(row0, 8), pl.ds(start, 128)],
            gbuf.at[j],
            sem,
        )
        d.start()
        descs.append(d)
    for d in descs:
        d.wait()

    tsv = tsc_ref[...]
    g = gbuf[...]
    sub = lax.broadcasted_iota(jnp.int32, g.shape, 1)
    lane = lax.broadcasted_iota(jnp.int32, g.shape, 2)
    myrow = lax.broadcasted_iota(jnp.int32, g.shape, 0) & 7
    off = jnp.bitwise_and(tsv, 127)[:, :, None]
    vt_tile = jnp.sum(
        jnp.where((sub == myrow) & (lane == off), g, 0.0), axis=(1, 2)
    )[:, None]

    strip = strip_ref[...]
    scols = va + lax.broadcasted_iota(jnp.int32, strip.shape, 1)
    vt_strip = jnp.sum(
        jnp.where(scols == tsv, strip, 0.0), axis=1, keepdims=True
    )
    strip_rs = jnp.sum(strip, axis=1, keepdims=True)
    vt_all = vt_strip + jnp.where(tsv < va, vt_tile, 0.0)
    p += jnp.sum(
        jnp.where(
            tsv != _PAD, c + (eps - conf) * vt_all - eps * strip_rs, 0.0
        )
    )

    @pl.when(s == 0)
    def _():
        out_ref[0, 0] = 0.0

    out_ref[0, 0] += p


def _tc_c_body(p_ref, s1_ref, s0_ref, tsc_ref, out_ref, *, eps):
    rs = jnp.sum(s1_ref[...], axis=1, keepdims=True)
    vb = s0_ref[...][:, _PAD:_PAD + 1]
    m = tsc_ref[...] != _PAD
    out_ref[0, 0] = p_ref[0, 0] + jnp.sum(
        jnp.where(m, -eps * rs + eps * vb, 0.0)
    )


def kernel(log_probs, targets):
    lp = log_probs.reshape(-1, log_probs.shape[-1])
    n, v = lp.shape
    tgt = targets.reshape(-1, 1).astype(jnp.int32)
    nsc = n - _RT
    rb = 64
    eps = _SMOOTHING / (v - 2)
    conf = 1.0 - _SMOOTHING
    c = (v - 2) * eps * math.log(eps) + conf * math.log(conf)

    sc_rowsum = pl.kernel(
        functools.partial(_sc_body, n=n, v=v),
        out_type=(
            jax.ShapeDtypeStruct((nsc, 16), jnp.float32),
            jax.ShapeDtypeStruct((nsc, 16), jnp.float32),
        ),
        mesh=plsc.VectorSubcoreMesh(core_axis_name="c", subcore_axis_name="s"),
        scratch_types=[
            pltpu.VMEM((nsc // _NW, _CH), jnp.float32),
            pltpu.VMEM((nsc // _NW, 16), jnp.float32),
            pltpu.VMEM((nsc // _NW, 16), jnp.float32),
        ],
        compiler_params=pltpu.CompilerParams(use_tc_tiling_on_sc=True),
    )
    s1, s0 = sc_rowsum(lp)

    p_a = pl.pallas_call(
        functools.partial(_tc_a_body, eps=eps, conf=conf, c=c, v=v, rb=rb),
        grid=(_RT // rb,),
        in_specs=[
            pl.BlockSpec((rb, 1), lambda i: (i, 0)),
            pl.BlockSpec((rb, 1), lambda i: (i, 0), memory_space=pltpu.SMEM),
            pl.BlockSpec((rb, 1), lambda i: (i, 0)),
            pl.BlockSpec((rb, v), lambda i: (i, 0)),
            pl.BlockSpec((rb, v % 128), lambda i: (i, 0)),
            pl.BlockSpec(memory_space=pl.ANY),
        ],
        out_specs=pl.BlockSpec(
            (1, 1), lambda i: (0, 0), memory_space=pltpu.SMEM
        ),
        out_shape=jax.ShapeDtypeStruct((1, 1), jnp.float32),
        scratch_shapes=[
            pltpu.VMEM((rb, 8, 128), jnp.float32),
            pltpu.SemaphoreType.DMA,
        ],
    )(tgt[:_RT], tgt[_RT:], tgt[_RT:], lp, lp[_RT:, (v // 128) * 128:], lp)

    out = pl.pallas_call(
        functools.partial(_tc_c_body, eps=eps),
        in_specs=[
            pl.BlockSpec(memory_space=pltpu.SMEM),
            pl.BlockSpec(memory_space=pltpu.VMEM),
            pl.BlockSpec(memory_space=pltpu.VMEM),
            pl.BlockSpec(memory_space=pltpu.VMEM),
        ],
        out_specs=pl.BlockSpec(memory_space=pltpu.SMEM),
        out_shape=jax.ShapeDtypeStruct((1, 1), jnp.float32),
    )(p_a, s1, s0, tgt[_RT:])
    return out[0, 0]


# speedup vs baseline: 3.2953x; 3.2953x over previous
import functools
import math

import jax
import jax.numpy as jnp
from jax import lax
from jax.experimental import pallas as pl
from jax.experimental.pallas import tpu as pltpu
from jax.experimental.pallas import tpu_sc as plsc

_SMOOTHING = 0.1
_PAD = 1

_NC = 2
_NS = 16
_NW = _NC * _NS

_VT = 64000
_G = 16
_SCCH = 600


def _sc_body(lpt_hbm, s1_hbm, buf, obuf, *, v, n):
    wid = lax.axis_index("s") * _NC + lax.axis_index("c")
    a = wid // 8
    b = wid % 8
    nrows = (v - _VT) // 4
    r0 = _VT + a * nrows
    c0 = b * 128
    zero = jnp.zeros((16,), jnp.float32)
    acc = tuple([zero] * 8)
    for ci in range(nrows // _SCCH):
        pltpu.sync_copy(
            lpt_hbm.at[pl.ds(r0 + ci * _SCCH, _SCCH), pl.ds(c0, 128)], buf
        )

        def body(j, acc_):
            return tuple(
                acc_[l] + buf[j, pl.ds(l * 16, 16)] for l in range(8)
            )

        acc = lax.fori_loop(0, _SCCH, body, acc)
    for l in range(8):
        obuf[l, :] = acc[l]
    pltpu.sync_copy(obuf, s1_hbm.at[wid])


def _tc_a_body(
    tgt_ref, tsc_s_ref, tsc_v_ref, lpt_ref, lpt_any, out_ref,
    acc_s, acc_vt, vbrow, gbuf, sem, *, eps, conf, c, v, rv, gpb
):
    s = pl.program_id(0)

    @pl.when(s == 0)
    def _():
        out_ref[0, 0] = 0.0
        acc_s[...] = jnp.zeros_like(acc_s)
        acc_vt[...] = jnp.zeros_like(acc_vt)

    blk = lpt_ref[...]
    tt = tgt_ref[...]
    acc_s[...] += jnp.sum(blk, axis=0, keepdims=True)
    rows = s * rv + lax.broadcasted_iota(jnp.int32, blk.shape, 0)
    acc_vt[...] += jnp.sum(
        jnp.where(rows == tt, blk, 0.0), axis=0, keepdims=True
    )

    @pl.when(s == 0)
    def _():
        vbrow[...] = blk[_PAD:_PAD + 1, :]

    descs = []
    for j in range(gpb):
        tj = tsc_s_ref[j, 0]
        rowbase = pl.multiple_of((tj >> 3) << 3, 8)
        jg = s * gpb + j
        colbase = pl.multiple_of((jg >> 7) << 7, 128)
        d = pltpu.make_async_copy(
            lpt_any.at[pl.ds(rowbase, 8), pl.ds(colbase, 128)],
            gbuf.at[j],
            sem,
        )
        d.start()
        descs.append(d)
    for d in descs:
        d.wait()

    tsv = tsc_v_ref[...]
    g = gbuf[...]
    sub = lax.broadcasted_iota(jnp.int32, g.shape, 1)
    lane = lax.broadcasted_iota(jnp.int32, g.shape, 2)
    want_sub = jnp.bitwise_and(tsv, 7)[:, :, None]
    want_lane = ((s & 1) * gpb + lax.broadcasted_iota(
        jnp.int32, (gpb, 1), 0
    ))[:, :, None]
    val = jnp.sum(
        jnp.where((sub == want_sub) & (lane == want_lane), g, 0.0),
        axis=(1, 2),
    )[:, None]
    out_ref[0, 0] += jnp.sum(
        jnp.where((tsv >= _VT) & (tsv != _PAD), (eps - conf) * val, 0.0)
    )

    @pl.when(s == pl.num_programs(0) - 1)
    def _():
        m = tt != _PAD
        out_ref[0, 0] += jnp.sum(
            jnp.where(
                m,
                c - eps * acc_s[...] + eps * vbrow[...]
                + (eps - conf) * acc_vt[...],
                0.0,
            )
        )


def _tc_c_body(p_ref, s1_ref, w_ref, out_ref):
    out_ref[0, 0] = p_ref[0, 0] + jnp.sum(s1_ref[...] * w_ref[...])


def kernel(log_probs, targets):
    lp = log_probs.reshape(-1, log_probs.shape[-1])
    n, v = lp.shape
    lpt = lp.T
    tgt = targets.reshape(-1).astype(jnp.int32)
    rv = _VT // _G
    gpb = n // _G
    eps = _SMOOTHING / (v - 2)
    conf = 1.0 - _SMOOTHING
    c = (v - 2) * eps * math.log(eps) + conf * math.log(conf)

    sc_colsum = pl.kernel(
        functools.partial(_sc_body, v=v, n=n),
        out_type=jax.ShapeDtypeStruct((_NW, 8, 16), jnp.float32),
        mesh=plsc.VectorSubcoreMesh(core_axis_name="c", subcore_axis_name="s"),
        scratch_types=[
            pltpu.VMEM((_SCCH, 128), jnp.float32),
            pltpu.VMEM((8, 16), jnp.float32),
        ],
        compiler_params=pltpu.CompilerParams(use_tc_tiling_on_sc=True),
    )
    s1 = sc_colsum(lpt)

    p_a = pl.pallas_call(
        functools.partial(
            _tc_a_body, eps=eps, conf=conf, c=c, v=v, rv=rv, gpb=gpb
        ),
        grid=(_G,),
        in_specs=[
            pl.BlockSpec((1, n), lambda i: (0, 0)),
            pl.BlockSpec((gpb, 1), lambda i: (i, 0), memory_space=pltpu.SMEM),
            pl.BlockSpec((gpb, 1), lambda i: (i, 0)),
            pl.BlockSpec((rv, n), lambda i: (i, 0)),
            pl.BlockSpec(memory_space=pl.ANY),
        ],
        out_specs=pl.BlockSpec(
            (1, 1), lambda i: (0, 0), memory_space=pltpu.SMEM
        ),
        out_shape=jax.ShapeDtypeStruct((1, 1), jnp.float32),
        scratch_shapes=[
            pltpu.VMEM((1, n), jnp.float32),
            pltpu.VMEM((1, n), jnp.float32),
            pltpu.VMEM((1, n), jnp.float32),
            pltpu.VMEM((gpb, 8, 128), jnp.float32),
            pltpu.SemaphoreType.DMA,
        ],
    )(tgt.reshape(1, n), tgt.reshape(n, 1), tgt.reshape(n, 1), lpt, lpt)

    w = jnp.where(tgt != _PAD, -eps, 0.0).reshape(1, 8, 8, 16)
    w = jnp.broadcast_to(w, (4, 8, 8, 16)).reshape(_NW, 8, 16)

    out = pl.pallas_call(
        _tc_c_body,
        in_specs=[
            pl.BlockSpec(memory_space=pltpu.SMEM),
            pl.BlockSpec(memory_space=pltpu.VMEM),
            pl.BlockSpec(memory_space=pltpu.VMEM),
        ],
        out_specs=pl.BlockSpec(memory_space=pltpu.SMEM),
        out_shape=jax.ShapeDtypeStruct((1, 1), jnp.float32),
    )(p_a, s1, w)
    return out[0, 0]
